# Initial kernel scaffold; baseline (speedup 1.0000x reference)
import functools

import jax
import jax.numpy as jnp
from jax import lax
from jax.experimental import pallas as pl
from jax.experimental.pallas import tpu as pltpu
from jax.experimental.pallas import tpu_sc as plsc

N = 50000
X_DIM = 128
H_DIM = 64
G_DIM = 32
E = 1600000

NC = 2
NS = 16
NW = NC * NS

CHUNK = 128
CHUNKS = 392
T_EDGES = CHUNKS * CHUNK
EP = NW * T_EDGES
NP = 51200
ROWS_PER_TILE = NP // NS

RB = 2000
GRID = N // RB


def _tc1_body(x_ref, t_ref, w1_ref, b1_ref, wg_ref, phi_ref, xw_ref):
    x = x_ref[...]
    phi = jnp.maximum(
        jax.lax.dot_general(x, w1_ref[...], (((1,), (0,)), ((), ())),
                            preferred_element_type=jnp.float32)
        + b1_ref[...], 0.0)
    phi_ref[...] = phi
    phi_t = t_ref[...] * phi
    xw_ref[...] = jax.lax.dot_general(phi_t, wg_ref[...], (((1,), (0,)), ((), ())),
                                      preferred_element_type=jnp.float32)


def _tc1(features, t2d, W1, b1r, Wg):
    return pl.pallas_call(
        _tc1_body,
        grid=(GRID,),
        in_specs=[
            pl.BlockSpec((RB, X_DIM), lambda i: (i, 0)),
            pl.BlockSpec((RB, 1), lambda i: (i, 0)),
            pl.BlockSpec((X_DIM, H_DIM), lambda i: (0, 0)),
            pl.BlockSpec((1, H_DIM), lambda i: (0, 0)),
            pl.BlockSpec((H_DIM, G_DIM), lambda i: (0, 0)),
        ],
        out_specs=[
            pl.BlockSpec((RB, H_DIM), lambda i: (i, 0)),
            pl.BlockSpec((RB, G_DIM), lambda i: (i, 0)),
        ],
        out_shape=[
            jax.ShapeDtypeStruct((N, H_DIM), jnp.float32),
            jax.ShapeDtypeStruct((N, G_DIM), jnp.float32),
        ],
    )(features, t2d, W1, b1r, Wg)


def _sca_body(dst_hbm, out_hbm, dst_v, hist):
    c = lax.axis_index("c")
    s = lax.axis_index("s")
    w = c * NS + s
    pltpu.sync_copy(dst_hbm.at[w], dst_v)

    def zero_body(i, carry):
        hist[pl.ds(i * 16, 16)] = jnp.zeros((16,), jnp.float32)
        return carry

    lax.fori_loop(0, NP // 16, zero_body, 0)
    ones = jnp.ones((16,), jnp.float32)

    def chunk_body(j, carry):
        for v in range(CHUNK // 16):
            idx = dst_v[j, pl.ds(v * 16, 16)]
            plsc.addupdate_scatter(hist, [idx], ones)
        return carry

    lax.fori_loop(0, CHUNKS, chunk_body, 0)
    pltpu.sync_copy(hist, out_hbm.at[w])


def _sca(dst_p):
    mesh = plsc.VectorSubcoreMesh(core_axis_name="c", subcore_axis_name="s")
    return pl.kernel(
        _sca_body,
        out_type=jax.ShapeDtypeStruct((NW, NP), jnp.float32),
        mesh=mesh,
        scratch_types=[
            pltpu.VMEM((CHUNKS, CHUNK), jnp.int32),
            pltpu.VMEM((NP,), jnp.float32),
        ],
    )(dst_p)


def _tc2_body(degp_ref, xw_ref, ones_ref, xs_ref, dinv_ref):
    degsum = jax.lax.dot_general(degp_ref[...], ones_ref[...],
                                 (((0,), (0,)), ((), ())),
                                 preferred_element_type=jnp.float32)
    dinv = jax.lax.rsqrt(degsum + 1.0)
    dinv_ref[...] = dinv
    xs_ref[...] = xw_ref[...] * dinv


def _tc2(deg_parts, xw, ones_col):
    return pl.pallas_call(
        _tc2_body,
        grid=(GRID,),
        in_specs=[
            pl.BlockSpec((NW, RB), lambda i: (0, i)),
            pl.BlockSpec((RB, G_DIM), lambda i: (i, 0)),
            pl.BlockSpec((NW, 1), lambda i: (0, 0)),
        ],
        out_specs=[
            pl.BlockSpec((RB, G_DIM), lambda i: (i, 0)),
            pl.BlockSpec((RB, 1), lambda i: (i, 0)),
        ],
        out_shape=[
            jax.ShapeDtypeStruct((N, G_DIM), jnp.float32),
            jax.ShapeDtypeStruct((N, 1), jnp.float32),
        ],
    )(deg_parts, xw, ones_col)


def _scb_body(xs_hbm, src_hbm, dst_hbm, zeros_hbm, out0, out1,
              src_v, dst_v, rows, acc, gsem):
    c = lax.axis_index("c")
    s = lax.axis_index("s")
    w = c * NS + s
    base = s * ROWS_PER_TILE
    sl = pl.ds(base, ROWS_PER_TILE)
    pltpu.sync_copy(zeros_hbm, acc.at[sl])
    pltpu.sync_copy(src_hbm.at[w], src_v)
    pltpu.sync_copy(dst_hbm.at[w], dst_v)
    plsc.subcore_barrier()

    def chunk_body(j, carry):
        pltpu.async_copy(xs_hbm.at[src_v.at[j]], rows, gsem).wait()
        pltpu.sync_copy(rows, acc.at[dst_v.at[j]], add=True)
        return carry

    lax.fori_loop(0, CHUNKS, chunk_body, 0)
    plsc.subcore_barrier()

    @pl.when(c == 0)
    def _():
        pltpu.sync_copy(acc.at[sl], out0.at[sl])

    @pl.when(c == 1)
    def _():
        pltpu.sync_copy(acc.at[sl], out1.at[sl])


def _scb(xs, src_p, dst_p, zeros_blk):
    mesh = plsc.VectorSubcoreMesh(core_axis_name="c", subcore_axis_name="s")
    return pl.kernel(
        _scb_body,
        out_type=[
            jax.ShapeDtypeStruct((NP, G_DIM), jnp.float32),
            jax.ShapeDtypeStruct((NP, G_DIM), jnp.float32),
        ],
        mesh=mesh,
        scratch_types=[
            pltpu.VMEM((CHUNKS, CHUNK), jnp.int32),
            pltpu.VMEM((CHUNKS, CHUNK), jnp.int32),
            pltpu.VMEM((CHUNK, G_DIM), jnp.float32),
            pltpu.VMEM_SHARED((NP, G_DIM), jnp.float32),
            pltpu.SemaphoreType.DMA,
        ],
    )(xs, src_p, dst_p, zeros_blk)


def _tc3_body(phi_ref, xs_ref, dinv_ref, a0_ref, a1_ref, bg_ref,
              w00a_ref, w00b_ref, b00_ref, w10a_ref, w10b_ref, b10_ref,
              wt0_ref, bt0_ref, wt1_ref, bt1_ref, y0_ref, y1_ref):
    gnn = dinv_ref[...] * (a0_ref[...] + a1_ref[...] + xs_ref[...]) + bg_ref[...]
    phi = phi_ref[...]

    def head(wa, wb, b, wt, bt):
        y = jnp.maximum(
            jax.lax.dot_general(phi, wa, (((1,), (0,)), ((), ())),
                                preferred_element_type=jnp.float32)
            + jax.lax.dot_general(gnn, wb, (((1,), (0,)), ((), ())),
                                  preferred_element_type=jnp.float32)
            + b, 0.0)
        return jax.lax.dot_general(y, wt, (((1,), (0,)), ((), ())),
                                   preferred_element_type=jnp.float32) + bt

    y0_ref[...] = head(w00a_ref[...], w00b_ref[...], b00_ref[...],
                       wt0_ref[...], bt0_ref[...])
    y1_ref[...] = head(w10a_ref[...], w10b_ref[...], b10_ref[...],
                       wt1_ref[...], bt1_ref[...])


def _tc3(phi_x, xs, dinv, acc0, acc1, bgr, W00a, W00b, b00r,
         W10a, W10b, b10r, Wt01, bt01r, Wt11, bt11r):
    YR = H_DIM + G_DIM

    def full(shape):
        return pl.BlockSpec(shape, lambda i: tuple(0 for _ in shape))

    return pl.pallas_call(
        _tc3_body,
        grid=(GRID,),
        in_specs=[
            pl.BlockSpec((RB, H_DIM), lambda i: (i, 0)),
            pl.BlockSpec((RB, G_DIM), lambda i: (i, 0)),
            pl.BlockSpec((RB, 1), lambda i: (i, 0)),
            pl.BlockSpec((RB, G_DIM), lambda i: (i, 0)),
            pl.BlockSpec((RB, G_DIM), lambda i: (i, 0)),
            full((1, G_DIM)),
            full((H_DIM, YR)), full((G_DIM, YR)), full((1, YR)),
            full((H_DIM, YR)), full((G_DIM, YR)), full((1, YR)),
            full((YR, 1)), full((1, 1)),
            full((YR, 1)), full((1, 1)),
        ],
        out_specs=[
            pl.BlockSpec((RB, 1), lambda i: (i, 0)),
            pl.BlockSpec((RB, 1), lambda i: (i, 0)),
        ],
        out_shape=[
            jax.ShapeDtypeStruct((N, 1), jnp.float32),
            jax.ShapeDtypeStruct((N, 1), jnp.float32),
        ],
    )(phi_x, xs, dinv, acc0, acc1, bgr, W00a, W00b, b00r,
      W10a, W10b, b10r, Wt01, bt01r, Wt11, bt11r)


def kernel(features, treatments, edge_index, W1, b1, Wg, bg,
           W00, b00, W10, b10, Wt01, bt01, Wt11, bt11):
    t2d = treatments.reshape(N, 1)
    b1r = b1.reshape(1, H_DIM)
    bgr = bg.reshape(1, G_DIM)
    b00r = b00.reshape(1, -1)
    b10r = b10.reshape(1, -1)
    bt01r = bt01.reshape(1, 1)
    bt11r = bt11.reshape(1, 1)
    W00a, W00b = W00[:H_DIM], W00[H_DIM:]
    W10a, W10b = W10[:H_DIM], W10[H_DIM:]

    pad = EP - E
    src = edge_index[0]
    dst = edge_index[1]
    pad_src = (jnp.arange(pad, dtype=jnp.int32) * 8191) % N
    pad_dst = jnp.full((pad,), N, dtype=jnp.int32)
    src_p = jnp.concatenate([src, pad_src]).reshape(NW, CHUNKS, CHUNK)
    dst_p = jnp.concatenate([dst, pad_dst]).reshape(NW, CHUNKS, CHUNK)

    zeros_blk = jnp.zeros((ROWS_PER_TILE, G_DIM), jnp.float32)
    ones_col = jnp.ones((NW, 1), jnp.float32)

    phi_x, xw = _tc1(features, t2d, W1, b1r, Wg)
    deg_parts = _sca(dst_p)
    xs, dinv = _tc2(deg_parts, xw, ones_col)
    acc0, acc1 = _scb(xs, src_p, dst_p, zeros_blk)
    y0_2d, y1_2d = _tc3(phi_x, xs, dinv, acc0[:N], acc1[:N], bgr,
                        W00a, W00b, b00r, W10a, W10b, b10r,
                        Wt01, bt01r, Wt11, bt11r)
    return (y1_2d.reshape(-1), y0_2d.reshape(-1), phi_x)


# speedup vs baseline: 45.0421x; 45.0421x over previous
import functools

import jax
import jax.numpy as jnp
from jax import lax
from jax.experimental import pallas as pl
from jax.experimental.pallas import tpu as pltpu
from jax.experimental.pallas import tpu_sc as plsc

N = 50000
X_DIM = 128
H_DIM = 64
G_DIM = 32
E = 1600000

NC = 2
NS = 16
NW = NC * NS

CHUNK = 128
CHUNKS = 392
T_EDGES = CHUNKS * CHUNK
EP = NW * T_EDGES
NP = 51200
ROWS_PER_TILE = NP // NS

RB = 2048
GRID = NP // RB


def _tc1_body(x_ref, t_ref, w1_ref, b1_ref, wg_ref, phi_ref, xw_ref):
    x = x_ref[...]
    phi = jnp.maximum(
        jax.lax.dot_general(x, w1_ref[...], (((1,), (0,)), ((), ())),
                            preferred_element_type=jnp.float32)
        + b1_ref[...], 0.0)
    phi_ref[...] = phi
    phi_t = t_ref[...] * phi
    xw_ref[...] = jax.lax.dot_general(phi_t, wg_ref[...], (((1,), (0,)), ((), ())),
                                      preferred_element_type=jnp.float32)


def _tc1(features, t2d, W1, b1r, Wg):
    return pl.pallas_call(
        _tc1_body,
        grid=(GRID,),
        in_specs=[
            pl.BlockSpec((RB, X_DIM), lambda i: (i, 0)),
            pl.BlockSpec((RB, 1), lambda i: (i, 0)),
            pl.BlockSpec((X_DIM, H_DIM), lambda i: (0, 0)),
            pl.BlockSpec((1, H_DIM), lambda i: (0, 0)),
            pl.BlockSpec((H_DIM, G_DIM), lambda i: (0, 0)),
        ],
        out_specs=[
            pl.BlockSpec((RB, H_DIM), lambda i: (i, 0)),
            pl.BlockSpec((RB, G_DIM), lambda i: (i, 0)),
        ],
        out_shape=[
            jax.ShapeDtypeStruct((N, H_DIM), jnp.float32),
            jax.ShapeDtypeStruct((N, G_DIM), jnp.float32),
        ],
    )(features, t2d, W1, b1r, Wg)


def _sca_body(dst_hbm, out_hbm, dst_v, hist):
    c = lax.axis_index("c")
    s = lax.axis_index("s")
    w = c * NS + s
    pltpu.sync_copy(dst_hbm.at[w], dst_v)

    def zero_body(i, carry):
        hist[pl.ds(i * 16, 16)] = jnp.zeros((16,), jnp.float32)
        return carry

    lax.fori_loop(0, NP // 16, zero_body, 0)
    ones = jnp.ones((16,), jnp.float32)

    def chunk_body(j, carry):
        for v in range(CHUNK // 16):
            idx = dst_v[j, pl.ds(v * 16, 16)]
            plsc.addupdate_scatter(hist, [idx], ones)
        return carry

    lax.fori_loop(0, CHUNKS, chunk_body, 0)
    pltpu.sync_copy(hist, out_hbm.at[w])


def _sca(dst_p):
    mesh = plsc.VectorSubcoreMesh(core_axis_name="c", subcore_axis_name="s")
    return pl.kernel(
        _sca_body,
        out_type=jax.ShapeDtypeStruct((NW, NP), jnp.float32),
        mesh=mesh,
        compiler_params=pltpu.CompilerParams(needs_layout_passes=False),
        scratch_types=[
            pltpu.VMEM((CHUNKS, CHUNK), jnp.int32),
            pltpu.VMEM((NP,), jnp.float32),
        ],
    )(dst_p)


def _tc2_body(degp_ref, xw_ref, ones_ref, xs_ref, dinv_ref):
    degsum = jax.lax.dot_general(degp_ref[...], ones_ref[...],
                                 (((0,), (0,)), ((), ())),
                                 preferred_element_type=jnp.float32)
    dinv = jax.lax.rsqrt(degsum + 1.0)
    dinv_ref[...] = dinv
    xs_ref[...] = xw_ref[...] * dinv


def _tc2(deg_parts, xw, ones_col):
    return pl.pallas_call(
        _tc2_body,
        grid=(GRID,),
        in_specs=[
            pl.BlockSpec((NW, RB), lambda i: (0, i)),
            pl.BlockSpec((RB, G_DIM), lambda i: (i, 0)),
            pl.BlockSpec((NW, 1), lambda i: (0, 0)),
        ],
        out_specs=[
            pl.BlockSpec((RB, G_DIM), lambda i: (i, 0)),
            pl.BlockSpec((RB, 1), lambda i: (i, 0)),
        ],
        out_shape=[
            jax.ShapeDtypeStruct((N, G_DIM), jnp.float32),
            jax.ShapeDtypeStruct((N, 1), jnp.float32),
        ],
    )(deg_parts, xw, ones_col)


SUP = 8
GROUPS = CHUNKS // SUP


def _scb_body(xs_hbm, src_hbm, dst_hbm, zeros_hbm, out0, out1,
              src_v, dst_v, rows, acc, gsem):
    c = lax.axis_index("c")
    s = lax.axis_index("s")
    w = c * NS + s
    base = s * ROWS_PER_TILE
    sl = pl.ds(base, ROWS_PER_TILE)
    pltpu.sync_copy(zeros_hbm, acc.at[sl])
    plsc.subcore_barrier()

    def group_body(g, carry):
        pltpu.sync_copy(src_hbm.at[w].at[pl.ds(g * SUP, SUP)], src_v)
        pltpu.sync_copy(dst_hbm.at[w].at[pl.ds(g * SUP, SUP)], dst_v)
        for j in range(SUP):
            pltpu.async_copy(xs_hbm.at[src_v.at[j]], rows, gsem).wait()
            pltpu.sync_copy(rows, acc.at[dst_v.at[j]], add=True)
        return carry

    lax.fori_loop(0, GROUPS, group_body, 0)
    plsc.subcore_barrier()

    @pl.when(c == 0)
    def _():
        pltpu.sync_copy(acc.at[sl], out0.at[sl])

    @pl.when(c == 1)
    def _():
        pltpu.sync_copy(acc.at[sl], out1.at[sl])


def _scb(xs, src_p, dst_p, zeros_blk):
    mesh = plsc.VectorSubcoreMesh(core_axis_name="c", subcore_axis_name="s")
    return pl.kernel(
        _scb_body,
        out_type=[
            jax.ShapeDtypeStruct((NP, G_DIM), jnp.float32),
            jax.ShapeDtypeStruct((NP, G_DIM), jnp.float32),
        ],
        mesh=mesh,
        compiler_params=pltpu.CompilerParams(use_tc_tiling_on_sc=False),
        scratch_types=[
            pltpu.VMEM((SUP, CHUNK), jnp.int32),
            pltpu.VMEM((SUP, CHUNK), jnp.int32),
            pltpu.VMEM((CHUNK, G_DIM), jnp.float32),
            pltpu.VMEM_SHARED((NP, G_DIM), jnp.float32),
            pltpu.SemaphoreType.DMA,
        ],
    )(xs, src_p, dst_p, zeros_blk)


def _tc3_body(phi_ref, xs_ref, dinv_ref, a0_ref, a1_ref, bg_ref,
              w00a_ref, w00b_ref, b00_ref, w10a_ref, w10b_ref, b10_ref,
              wt0_ref, bt0_ref, wt1_ref, bt1_ref, y0_ref, y1_ref):
    gnn = dinv_ref[...] * (a0_ref[...] + a1_ref[...] + xs_ref[...]) + bg_ref[...]
    phi = phi_ref[...]

    def head(wa, wb, b, wt, bt):
        y = jnp.maximum(
            jax.lax.dot_general(phi, wa, (((1,), (0,)), ((), ())),
                                preferred_element_type=jnp.float32)
            + jax.lax.dot_general(gnn, wb, (((1,), (0,)), ((), ())),
                                  preferred_element_type=jnp.float32)
            + b, 0.0)
        return jax.lax.dot_general(y, wt, (((1,), (0,)), ((), ())),
                                   preferred_element_type=jnp.float32) + bt

    y0_ref[...] = head(w00a_ref[...], w00b_ref[...], b00_ref[...],
                       wt0_ref[...], bt0_ref[...])
    y1_ref[...] = head(w10a_ref[...], w10b_ref[...], b10_ref[...],
                       wt1_ref[...], bt1_ref[...])


def _tc3(phi_x, xs, dinv, acc0, acc1, bgr, W00a, W00b, b00r,
         W10a, W10b, b10r, Wt01, bt01r, Wt11, bt11r):
    YR = H_DIM + G_DIM

    def full(shape):
        return pl.BlockSpec(shape, lambda i: tuple(0 for _ in shape))

    return pl.pallas_call(
        _tc3_body,
        grid=(GRID,),
        in_specs=[
            pl.BlockSpec((RB, H_DIM), lambda i: (i, 0)),
            pl.BlockSpec((RB, G_DIM), lambda i: (i, 0)),
            pl.BlockSpec((RB, 1), lambda i: (i, 0)),
            pl.BlockSpec((RB, G_DIM), lambda i: (i, 0)),
            pl.BlockSpec((RB, G_DIM), lambda i: (i, 0)),
            full((1, G_DIM)),
            full((H_DIM, YR)), full((G_DIM, YR)), full((1, YR)),
            full((H_DIM, YR)), full((G_DIM, YR)), full((1, YR)),
            full((YR, 1)), full((1, 1)),
            full((YR, 1)), full((1, 1)),
        ],
        out_specs=[
            pl.BlockSpec((RB, 1), lambda i: (i, 0)),
            pl.BlockSpec((RB, 1), lambda i: (i, 0)),
        ],
        out_shape=[
            jax.ShapeDtypeStruct((N, 1), jnp.float32),
            jax.ShapeDtypeStruct((N, 1), jnp.float32),
        ],
    )(phi_x, xs, dinv, acc0, acc1, bgr, W00a, W00b, b00r,
      W10a, W10b, b10r, Wt01, bt01r, Wt11, bt11r)


def kernel(features, treatments, edge_index, W1, b1, Wg, bg,
           W00, b00, W10, b10, Wt01, bt01, Wt11, bt11):
    t2d = treatments.reshape(N, 1)
    b1r = b1.reshape(1, H_DIM)
    bgr = bg.reshape(1, G_DIM)
    b00r = b00.reshape(1, -1)
    b10r = b10.reshape(1, -1)
    bt01r = bt01.reshape(1, 1)
    bt11r = bt11.reshape(1, 1)
    W00a, W00b = W00[:H_DIM], W00[H_DIM:]
    W10a, W10b = W10[:H_DIM], W10[H_DIM:]

    pad = EP - E
    src = edge_index[0]
    dst = edge_index[1]
    pad_src = (jnp.arange(pad, dtype=jnp.int32) * 8191) % N
    pad_dst = jnp.full((pad,), N, dtype=jnp.int32)
    src_p = jnp.concatenate([src, pad_src]).reshape(NW, CHUNKS, CHUNK)
    dst_p = jnp.concatenate([dst, pad_dst]).reshape(NW, CHUNKS, CHUNK)

    zeros_blk = jnp.zeros((ROWS_PER_TILE, G_DIM), jnp.float32)
    ones_col = jnp.ones((NW, 1), jnp.float32)

    phi_x, xw = _tc1(features, t2d, W1, b1r, Wg)
    deg_parts = _sca(dst_p)
    xs, dinv = _tc2(deg_parts, xw, ones_col)
    acc0, acc1 = _scb(xs, src_p, dst_p, zeros_blk)
    y0_2d, y1_2d = _tc3(phi_x, xs, dinv, acc0[:N], acc1[:N], bgr,
                        W00a, W00b, b00r, W10a, W10b, b10r,
                        Wt01, bt01r, Wt11, bt11r)
    return (y1_2d.reshape(-1), y0_2d.reshape(-1), phi_x)


# speedup vs baseline: 56.8547x; 1.2623x over previous
import functools

import jax
import jax.numpy as jnp
from jax import lax
from jax.experimental import pallas as pl
from jax.experimental.pallas import tpu as pltpu
from jax.experimental.pallas import tpu_sc as plsc

N = 50000
X_DIM = 128
H_DIM = 64
G_DIM = 32
E = 1600000

NC = 2
NS = 16
NW = NC * NS

CHUNK = 128
CHUNKS = 392
T_EDGES = CHUNKS * CHUNK
EP = NW * T_EDGES
NP = 51200
ROWS_PER_TILE = NP // NS

RB = 2048
GRID = NP // RB


def _tc1_body(x_ref, t_ref, w1_ref, b1_ref, wg_ref, phi_ref, xw_ref):
    x = x_ref[...]
    phi = jnp.maximum(
        jax.lax.dot_general(x, w1_ref[...], (((1,), (0,)), ((), ())),
                            preferred_element_type=jnp.float32)
        + b1_ref[...], 0.0)
    phi_ref[...] = phi
    phi_t = t_ref[...] * phi
    xw_ref[...] = jax.lax.dot_general(phi_t, wg_ref[...], (((1,), (0,)), ((), ())),
                                      preferred_element_type=jnp.float32)


def _tc1(features, t2d, W1, b1r, Wg):
    return pl.pallas_call(
        _tc1_body,
        grid=(GRID,),
        in_specs=[
            pl.BlockSpec((RB, X_DIM), lambda i: (i, 0)),
            pl.BlockSpec((RB, 1), lambda i: (i, 0)),
            pl.BlockSpec((X_DIM, H_DIM), lambda i: (0, 0)),
            pl.BlockSpec((1, H_DIM), lambda i: (0, 0)),
            pl.BlockSpec((H_DIM, G_DIM), lambda i: (0, 0)),
        ],
        out_specs=[
            pl.BlockSpec((RB, H_DIM), lambda i: (i, 0)),
            pl.BlockSpec((RB, G_DIM), lambda i: (i, 0)),
        ],
        out_shape=[
            jax.ShapeDtypeStruct((N, H_DIM), jnp.float32),
            jax.ShapeDtypeStruct((N, G_DIM), jnp.float32),
        ],
    )(features, t2d, W1, b1r, Wg)


def _sca_body(dst_hbm, out_hbm, dst_v, hist):
    c = lax.axis_index("c")
    s = lax.axis_index("s")
    w = c * NS + s
    pltpu.sync_copy(dst_hbm.at[w], dst_v)

    def zero_body(i, carry):
        hist[pl.ds(i * 16, 16)] = jnp.zeros((16,), jnp.float32)
        return carry

    lax.fori_loop(0, NP // 16, zero_body, 0)
    ones = jnp.ones((16,), jnp.float32)

    def chunk_body(j, carry):
        for v in range(CHUNK // 16):
            idx = dst_v[j, pl.ds(v * 16, 16)]
            plsc.addupdate_scatter(hist, [idx], ones)
        return carry

    lax.fori_loop(0, CHUNKS, chunk_body, 0)
    pltpu.sync_copy(hist, out_hbm.at[w])


def _sca(dst_p):
    mesh = plsc.VectorSubcoreMesh(core_axis_name="c", subcore_axis_name="s")
    return pl.kernel(
        _sca_body,
        out_type=jax.ShapeDtypeStruct((NW, NP), jnp.float32),
        mesh=mesh,
        compiler_params=pltpu.CompilerParams(needs_layout_passes=False),
        scratch_types=[
            pltpu.VMEM((CHUNKS, CHUNK), jnp.int32),
            pltpu.VMEM((NP,), jnp.float32),
        ],
    )(dst_p)


def _tc2_body(degp_ref, xw_ref, ones_ref, xs_ref, dinv_ref):
    degsum = jax.lax.dot_general(degp_ref[...], ones_ref[...],
                                 (((0,), (0,)), ((), ())),
                                 preferred_element_type=jnp.float32)
    dinv = jax.lax.rsqrt(degsum + 1.0)
    dinv_ref[...] = dinv
    xs_ref[...] = xw_ref[...] * dinv


def _tc2(deg_parts, xw, ones_col):
    return pl.pallas_call(
        _tc2_body,
        grid=(GRID,),
        in_specs=[
            pl.BlockSpec((NW, RB), lambda i: (0, i)),
            pl.BlockSpec((RB, G_DIM), lambda i: (i, 0)),
            pl.BlockSpec((NW, 1), lambda i: (0, 0)),
        ],
        out_specs=[
            pl.BlockSpec((RB, G_DIM), lambda i: (i, 0)),
            pl.BlockSpec((RB, 1), lambda i: (i, 0)),
        ],
        out_shape=[
            jax.ShapeDtypeStruct((N, G_DIM), jnp.float32),
            jax.ShapeDtypeStruct((N, 1), jnp.float32),
        ],
    )(deg_parts, xw, ones_col)


GRP = 8
GROUPS = CHUNKS // GRP


def _scb_body(xs_hbm, src_hbm, dst_hbm, zeros_hbm, out0, out1,
              src_v, dst_v, rows0, rows1,
              gsem0, gsem1, ssem0, ssem1, acc):
    c = lax.axis_index("c")
    s = lax.axis_index("s")
    w = c * NS + s
    base = s * ROWS_PER_TILE
    sl = pl.ds(base, ROWS_PER_TILE)
    pltpu.sync_copy(zeros_hbm, acc.at[sl])
    plsc.subcore_barrier()

    rows = (rows0, rows1)
    gsem = (gsem0, gsem1)
    ssem = (ssem0, ssem1)

    def group_body(g, carry):
        pltpu.sync_copy(src_hbm.at[w].at[pl.ds(g * GRP, GRP)], src_v)
        pltpu.sync_copy(dst_hbm.at[w].at[pl.ds(g * GRP, GRP)], dst_v)
        gath = [None] * GRP
        scat = [None] * GRP
        gath[0] = pltpu.async_copy(xs_hbm.at[src_v.at[0]], rows[0], gsem[0])
        gath[1] = pltpu.async_copy(xs_hbm.at[src_v.at[1]], rows[1], gsem[1])
        for j in range(GRP):
            par = j & 1
            gath[j].wait()
            scat[j] = pltpu.async_copy(rows[par], acc.at[dst_v.at[j]],
                                       ssem[par], add=True)
            if j + 2 < GRP:
                scat[j].wait()
                gath[j + 2] = pltpu.async_copy(
                    xs_hbm.at[src_v.at[j + 2]], rows[par], gsem[par])
        scat[GRP - 2].wait()
        scat[GRP - 1].wait()
        return carry

    lax.fori_loop(0, GROUPS, group_body, 0)
    plsc.subcore_barrier()

    @pl.when(c == 0)
    def _():
        pltpu.sync_copy(acc.at[sl], out0.at[sl])

    @pl.when(c == 1)
    def _():
        pltpu.sync_copy(acc.at[sl], out1.at[sl])


def _scb(xs, src_p, dst_p, zeros_blk):
    mesh = plsc.VectorSubcoreMesh(core_axis_name="c", subcore_axis_name="s")
    return pl.kernel(
        _scb_body,
        out_type=[
            jax.ShapeDtypeStruct((NP, G_DIM), jnp.float32),
            jax.ShapeDtypeStruct((NP, G_DIM), jnp.float32),
        ],
        mesh=mesh,
        compiler_params=pltpu.CompilerParams(use_tc_tiling_on_sc=False),
        scratch_types=[
            pltpu.VMEM((GRP, CHUNK), jnp.int32),
            pltpu.VMEM((GRP, CHUNK), jnp.int32),
            pltpu.VMEM((CHUNK, G_DIM), jnp.float32),
            pltpu.VMEM((CHUNK, G_DIM), jnp.float32),
            pltpu.SemaphoreType.DMA,
            pltpu.SemaphoreType.DMA,
            pltpu.SemaphoreType.DMA,
            pltpu.SemaphoreType.DMA,
            pltpu.VMEM_SHARED((NP, G_DIM), jnp.float32),
        ],
    )(xs, src_p, dst_p, zeros_blk)


def _tc3_body(phi_ref, xs_ref, dinv_ref, a0_ref, a1_ref, bg_ref,
              w00a_ref, w00b_ref, b00_ref, w10a_ref, w10b_ref, b10_ref,
              wt0_ref, bt0_ref, wt1_ref, bt1_ref, y0_ref, y1_ref):
    gnn = dinv_ref[...] * (a0_ref[...] + a1_ref[...] + xs_ref[...]) + bg_ref[...]
    phi = phi_ref[...]

    def head(wa, wb, b, wt, bt):
        y = jnp.maximum(
            jax.lax.dot_general(phi, wa, (((1,), (0,)), ((), ())),
                                preferred_element_type=jnp.float32)
            + jax.lax.dot_general(gnn, wb, (((1,), (0,)), ((), ())),
                                  preferred_element_type=jnp.float32)
            + b, 0.0)
        return jax.lax.dot_general(y, wt, (((1,), (0,)), ((), ())),
                                   preferred_element_type=jnp.float32) + bt

    y0_ref[...] = head(w00a_ref[...], w00b_ref[...], b00_ref[...],
                       wt0_ref[...], bt0_ref[...])
    y1_ref[...] = head(w10a_ref[...], w10b_ref[...], b10_ref[...],
                       wt1_ref[...], bt1_ref[...])


def _tc3(phi_x, xs, dinv, acc0, acc1, bgr, W00a, W00b, b00r,
         W10a, W10b, b10r, Wt01, bt01r, Wt11, bt11r):
    YR = H_DIM + G_DIM

    def full(shape):
        return pl.BlockSpec(shape, lambda i: tuple(0 for _ in shape))

    return pl.pallas_call(
        _tc3_body,
        grid=(GRID,),
        in_specs=[
            pl.BlockSpec((RB, H_DIM), lambda i: (i, 0)),
            pl.BlockSpec((RB, G_DIM), lambda i: (i, 0)),
            pl.BlockSpec((RB, 1), lambda i: (i, 0)),
            pl.BlockSpec((RB, G_DIM), lambda i: (i, 0)),
            pl.BlockSpec((RB, G_DIM), lambda i: (i, 0)),
            full((1, G_DIM)),
            full((H_DIM, YR)), full((G_DIM, YR)), full((1, YR)),
            full((H_DIM, YR)), full((G_DIM, YR)), full((1, YR)),
            full((YR, 1)), full((1, 1)),
            full((YR, 1)), full((1, 1)),
        ],
        out_specs=[
            pl.BlockSpec((RB, 1), lambda i: (i, 0)),
            pl.BlockSpec((RB, 1), lambda i: (i, 0)),
        ],
        out_shape=[
            jax.ShapeDtypeStruct((N, 1), jnp.float32),
            jax.ShapeDtypeStruct((N, 1), jnp.float32),
        ],
    )(phi_x, xs, dinv, acc0, acc1, bgr, W00a, W00b, b00r,
      W10a, W10b, b10r, Wt01, bt01r, Wt11, bt11r)


def kernel(features, treatments, edge_index, W1, b1, Wg, bg,
           W00, b00, W10, b10, Wt01, bt01, Wt11, bt11):
    t2d = treatments.reshape(N, 1)
    b1r = b1.reshape(1, H_DIM)
    bgr = bg.reshape(1, G_DIM)
    b00r = b00.reshape(1, -1)
    b10r = b10.reshape(1, -1)
    bt01r = bt01.reshape(1, 1)
    bt11r = bt11.reshape(1, 1)
    W00a, W00b = W00[:H_DIM], W00[H_DIM:]
    W10a, W10b = W10[:H_DIM], W10[H_DIM:]

    pad = EP - E
    src = edge_index[0]
    dst = edge_index[1]
    pad_src = (jnp.arange(pad, dtype=jnp.int32) * 8191) % N
    pad_dst = jnp.full((pad,), N, dtype=jnp.int32)
    src_p = jnp.concatenate([src, pad_src]).reshape(NW, CHUNKS, CHUNK)
    dst_p = jnp.concatenate([dst, pad_dst]).reshape(NW, CHUNKS, CHUNK)

    zeros_blk = jnp.zeros((ROWS_PER_TILE, G_DIM), jnp.float32)
    ones_col = jnp.ones((NW, 1), jnp.float32)

    phi_x, xw = _tc1(features, t2d, W1, b1r, Wg)
    deg_parts = _sca(dst_p)
    xs, dinv = _tc2(deg_parts, xw, ones_col)
    acc0, acc1 = _scb(xs, src_p, dst_p, zeros_blk)
    y0_2d, y1_2d = _tc3(phi_x, xs, dinv, acc0[:N], acc1[:N], bgr,
                        W00a, W00b, b00r, W10a, W10b, b10r,
                        Wt01, bt01r, Wt11, bt11r)
    return (y1_2d.reshape(-1), y0_2d.reshape(-1), phi_x)


# speedup vs baseline: 62.8650x; 1.1057x over previous
import jax
import jax.numpy as jnp
from jax import lax
from jax.experimental import pallas as pl
from jax.experimental.pallas import tpu as pltpu
from jax.experimental.pallas import tpu_sc as plsc

N = 50000
X_DIM = 128
H_DIM = 64
G_DIM = 32
E = 1600000

NC = 2
NS = 16
NW = NC * NS

CHUNK = 128
ECH = E // CHUNK
BASE_CH = ECH // NW
EXTRA = ECH - NW * BASE_CH
EXTRA_BASE = NW * BASE_CH

NP = 51200
ROWS_PER_TILE = NP // NS
DROW = 16
DROWS = NP // DROW
DROWS_PER_TILE = DROWS // NS
DCH = DROWS // CHUNK

RB = 2048
GRID = NP // RB

GRP = 6
GROUPS = BASE_CH // GRP


def _tc1_body(x_ref, t_ref, w1_ref, b1_ref, wg_ref, phi_ref, xw_ref):
    x = x_ref[...]
    phi = jnp.maximum(
        jax.lax.dot_general(x, w1_ref[...], (((1,), (0,)), ((), ())),
                            preferred_element_type=jnp.float32)
        + b1_ref[...], 0.0)
    phi_ref[...] = phi
    phi_t = t_ref[...] * phi
    xw_ref[...] = jax.lax.dot_general(phi_t, wg_ref[...], (((1,), (0,)), ((), ())),
                                      preferred_element_type=jnp.float32)


def _tc1(features, t2d, W1, b1r, Wg):
    return pl.pallas_call(
        _tc1_body,
        grid=(GRID,),
        in_specs=[
            pl.BlockSpec((RB, X_DIM), lambda i: (i, 0)),
            pl.BlockSpec((RB, 1), lambda i: (i, 0)),
            pl.BlockSpec((X_DIM, H_DIM), lambda i: (0, 0)),
            pl.BlockSpec((1, H_DIM), lambda i: (0, 0)),
            pl.BlockSpec((H_DIM, G_DIM), lambda i: (0, 0)),
        ],
        out_specs=[
            pl.BlockSpec((RB, H_DIM), lambda i: (i, 0)),
            pl.BlockSpec((RB, G_DIM), lambda i: (i, 0)),
        ],
        out_shape=[
            jax.ShapeDtypeStruct((N, H_DIM), jnp.float32),
            jax.ShapeDtypeStruct((N, G_DIM), jnp.float32),
        ],
    )(features, t2d, W1, b1r, Wg)


def _sca_body(ei_hbm, zeros16_hbm, idx_hbm, out_hbm,
              dst_v, dst_x, hist2, idx_v, deg_sh):
    c = lax.axis_index("c")
    s = lax.axis_index("s")
    w = c * NS + s
    start = w * BASE_CH
    pltpu.sync_copy(ei_hbm.at[1].at[pl.ds(start, BASE_CH)], dst_v)
    pltpu.sync_copy(idx_hbm, idx_v)
    pltpu.sync_copy(zeros16_hbm,
                    deg_sh.at[pl.ds(s * DROWS_PER_TILE, DROWS_PER_TILE)])

    z16 = jnp.zeros((16,), jnp.float32)

    def zero_body(i, carry):
        hist2[i, pl.ds(0, 16)] = z16
        return carry

    lax.fori_loop(0, DROWS, zero_body, 0)

    ones = jnp.ones((16,), jnp.float32)

    def hist_vec(idx):
        hi = lax.shift_right_logical(idx, 4)
        lo = lax.bitwise_and(idx, 15)
        plsc.addupdate_scatter(hist2, [hi, lo], ones)

    def chunk_body(j, carry):
        for v in range(CHUNK // 16):
            hist_vec(dst_v[j, pl.ds(v * 16, 16)])
        return carry

    lax.fori_loop(0, BASE_CH, chunk_body, 0)

    @pl.when(w < EXTRA)
    def _():
        pltpu.sync_copy(ei_hbm.at[1].at[pl.ds(EXTRA_BASE + w, 1)], dst_x)
        for v in range(CHUNK // 16):
            hist_vec(dst_x[0, pl.ds(v * 16, 16)])

    plsc.subcore_barrier()

    def red_body(k, carry):
        pltpu.sync_copy(hist2.at[pl.ds(k * CHUNK, CHUNK)],
                        deg_sh.at[idx_v.at[k]], add=True)
        return carry

    lax.fori_loop(0, DCH, red_body, 0)
    plsc.subcore_barrier()

    dsl = pl.ds(s * DROWS_PER_TILE, DROWS_PER_TILE)
    pltpu.sync_copy(deg_sh.at[dsl], out_hbm.at[c].at[dsl])


def _sca(ei3, zeros16, idx_rows):
    mesh = plsc.VectorSubcoreMesh(core_axis_name="c", subcore_axis_name="s")
    return pl.kernel(
        _sca_body,
        out_type=jax.ShapeDtypeStruct((NC, DROWS, DROW), jnp.float32),
        mesh=mesh,
        compiler_params=pltpu.CompilerParams(needs_layout_passes=False,
                                             use_tc_tiling_on_sc=False),
        scratch_types=[
            pltpu.VMEM((BASE_CH, CHUNK), jnp.int32),
            pltpu.VMEM((1, CHUNK), jnp.int32),
            pltpu.VMEM((DROWS, DROW), jnp.float32),
            pltpu.VMEM((DCH, CHUNK), jnp.int32),
            pltpu.VMEM_SHARED((DROWS, DROW), jnp.float32),
        ],
    )(ei3, zeros16, idx_rows)


def _tc2_body(degp_ref, xw_ref, ones_ref, xs_ref, dinv_ref):
    degsum = jax.lax.dot_general(degp_ref[...], ones_ref[...],
                                 (((0,), (0,)), ((), ())),
                                 preferred_element_type=jnp.float32)
    dinv = jax.lax.rsqrt(degsum + 1.0)
    dinv_ref[...] = dinv
    xs_ref[...] = xw_ref[...] * dinv


def _tc2(deg2, xw, ones_col):
    return pl.pallas_call(
        _tc2_body,
        grid=(GRID,),
        in_specs=[
            pl.BlockSpec((NC, RB), lambda i: (0, i)),
            pl.BlockSpec((RB, G_DIM), lambda i: (i, 0)),
            pl.BlockSpec((NC, 1), lambda i: (0, 0)),
        ],
        out_specs=[
            pl.BlockSpec((RB, G_DIM), lambda i: (i, 0)),
            pl.BlockSpec((RB, 1), lambda i: (i, 0)),
        ],
        out_shape=[
            jax.ShapeDtypeStruct((N, G_DIM), jnp.float32),
            jax.ShapeDtypeStruct((N, 1), jnp.float32),
        ],
    )(deg2, xw, ones_col)


def _scb_body(xs_hbm, ei_hbm, zeros_hbm, out0, out1,
              src_v, dst_v, src_x, dst_x, rows0, rows1,
              gsem0, gsem1, ssem0, ssem1, acc):
    c = lax.axis_index("c")
    s = lax.axis_index("s")
    w = c * NS + s
    base = s * ROWS_PER_TILE
    sl = pl.ds(base, ROWS_PER_TILE)
    pltpu.sync_copy(zeros_hbm, acc.at[sl])
    plsc.subcore_barrier()

    rows = (rows0, rows1)
    gsem = (gsem0, gsem1)
    ssem = (ssem0, ssem1)
    start = w * BASE_CH

    def group_body(g, carry):
        gsl = pl.ds(start + g * GRP, GRP)
        pltpu.sync_copy(ei_hbm.at[0].at[gsl], src_v)
        pltpu.sync_copy(ei_hbm.at[1].at[gsl], dst_v)
        gath = [None] * GRP
        scat = [None] * GRP
        gath[0] = pltpu.async_copy(xs_hbm.at[src_v.at[0]], rows[0], gsem[0])
        gath[1] = pltpu.async_copy(xs_hbm.at[src_v.at[1]], rows[1], gsem[1])
        for j in range(GRP):
            par = j & 1
            gath[j].wait()
            scat[j] = pltpu.async_copy(rows[par], acc.at[dst_v.at[j]],
                                       ssem[par], add=True)
            if j + 2 < GRP:
                scat[j].wait()
                gath[j + 2] = pltpu.async_copy(
                    xs_hbm.at[src_v.at[j + 2]], rows[par], gsem[par])
        scat[GRP - 2].wait()
        scat[GRP - 1].wait()
        return carry

    lax.fori_loop(0, GROUPS, group_body, 0)

    @pl.when(w < EXTRA)
    def _():
        xsl = pl.ds(EXTRA_BASE + w, 1)
        pltpu.sync_copy(ei_hbm.at[0].at[xsl], src_x)
        pltpu.sync_copy(ei_hbm.at[1].at[xsl], dst_x)
        pltpu.async_copy(xs_hbm.at[src_x.at[0]], rows0, gsem0).wait()
        pltpu.sync_copy(rows0, acc.at[dst_x.at[0]], add=True)

    plsc.subcore_barrier()

    @pl.when(c == 0)
    def _():
        pltpu.sync_copy(acc.at[sl], out0.at[sl])

    @pl.when(c == 1)
    def _():
        pltpu.sync_copy(acc.at[sl], out1.at[sl])


def _scb(xs, ei3, zeros_blk):
    mesh = plsc.VectorSubcoreMesh(core_axis_name="c", subcore_axis_name="s")
    return pl.kernel(
        _scb_body,
        out_type=[
            jax.ShapeDtypeStruct((NP, G_DIM), jnp.float32),
            jax.ShapeDtypeStruct((NP, G_DIM), jnp.float32),
        ],
        mesh=mesh,
        compiler_params=pltpu.CompilerParams(use_tc_tiling_on_sc=False),
        scratch_types=[
            pltpu.VMEM((GRP, CHUNK), jnp.int32),
            pltpu.VMEM((GRP, CHUNK), jnp.int32),
            pltpu.VMEM((1, CHUNK), jnp.int32),
            pltpu.VMEM((1, CHUNK), jnp.int32),
            pltpu.VMEM((CHUNK, G_DIM), jnp.float32),
            pltpu.VMEM((CHUNK, G_DIM), jnp.float32),
            pltpu.SemaphoreType.DMA,
            pltpu.SemaphoreType.DMA,
            pltpu.SemaphoreType.DMA,
            pltpu.SemaphoreType.DMA,
            pltpu.VMEM_SHARED((NP, G_DIM), jnp.float32),
        ],
    )(xs, ei3, zeros_blk)


def _tc3_body(phi_ref, xs_ref, dinv_ref, a0_ref, a1_ref, bg_ref,
              w00a_ref, w00b_ref, b00_ref, w10a_ref, w10b_ref, b10_ref,
              wt0_ref, bt0_ref, wt1_ref, bt1_ref, y0_ref, y1_ref):
    gnn = dinv_ref[...] * (a0_ref[...] + a1_ref[...] + xs_ref[...]) + bg_ref[...]
    phi = phi_ref[...]

    def head(wa, wb, b, wt, bt):
        y = jnp.maximum(
            jax.lax.dot_general(phi, wa, (((1,), (0,)), ((), ())),
                                preferred_element_type=jnp.float32)
            + jax.lax.dot_general(gnn, wb, (((1,), (0,)), ((), ())),
                                  preferred_element_type=jnp.float32)
            + b, 0.0)
        return jax.lax.dot_general(y, wt, (((1,), (0,)), ((), ())),
                                   preferred_element_type=jnp.float32) + bt

    y0_ref[...] = head(w00a_ref[...], w00b_ref[...], b00_ref[...],
                       wt0_ref[...], bt0_ref[...])
    y1_ref[...] = head(w10a_ref[...], w10b_ref[...], b10_ref[...],
                       wt1_ref[...], bt1_ref[...])


def _tc3(phi_x, xs, dinv, acc0, acc1, bgr, W00a, W00b, b00r,
         W10a, W10b, b10r, Wt01, bt01r, Wt11, bt11r):
    YR = H_DIM + G_DIM

    def full(shape):
        return pl.BlockSpec(shape, lambda i: tuple(0 for _ in shape))

    return pl.pallas_call(
        _tc3_body,
        grid=(GRID,),
        in_specs=[
            pl.BlockSpec((RB, H_DIM), lambda i: (i, 0)),
            pl.BlockSpec((RB, G_DIM), lambda i: (i, 0)),
            pl.BlockSpec((RB, 1), lambda i: (i, 0)),
            pl.BlockSpec((RB, G_DIM), lambda i: (i, 0)),
            pl.BlockSpec((RB, G_DIM), lambda i: (i, 0)),
            full((1, G_DIM)),
            full((H_DIM, YR)), full((G_DIM, YR)), full((1, YR)),
            full((H_DIM, YR)), full((G_DIM, YR)), full((1, YR)),
            full((YR, 1)), full((1, 1)),
            full((YR, 1)), full((1, 1)),
        ],
        out_specs=[
            pl.BlockSpec((RB, 1), lambda i: (i, 0)),
            pl.BlockSpec((RB, 1), lambda i: (i, 0)),
        ],
        out_shape=[
            jax.ShapeDtypeStruct((N, 1), jnp.float32),
            jax.ShapeDtypeStruct((N, 1), jnp.float32),
        ],
    )(phi_x, xs, dinv, acc0, acc1, bgr, W00a, W00b, b00r,
      W10a, W10b, b10r, Wt01, bt01r, Wt11, bt11r)


def kernel(features, treatments, edge_index, W1, b1, Wg, bg,
           W00, b00, W10, b10, Wt01, bt01, Wt11, bt11):
    t2d = treatments.reshape(N, 1)
    b1r = b1.reshape(1, H_DIM)
    bgr = bg.reshape(1, G_DIM)
    b00r = b00.reshape(1, -1)
    b10r = b10.reshape(1, -1)
    bt01r = bt01.reshape(1, 1)
    bt11r = bt11.reshape(1, 1)
    W00a, W00b = W00[:H_DIM], W00[H_DIM:]
    W10a, W10b = W10[:H_DIM], W10[H_DIM:]

    ei3 = edge_index.reshape(2, ECH, CHUNK)
    zeros_blk = jnp.zeros((ROWS_PER_TILE, G_DIM), jnp.float32)
    zeros16 = jnp.zeros((DROWS_PER_TILE, DROW), jnp.float32)
    idx_rows = jnp.arange(DROWS, dtype=jnp.int32).reshape(DCH, CHUNK)
    ones_col = jnp.ones((NC, 1), jnp.float32)

    phi_x, xw = _tc1(features, t2d, W1, b1r, Wg)
    deg2 = _sca(ei3, zeros16, idx_rows).reshape(NC, NP)
    xs, dinv = _tc2(deg2, xw, ones_col)
    acc0, acc1 = _scb(xs, ei3, zeros_blk)
    y0_2d, y1_2d = _tc3(phi_x, xs, dinv, acc0, acc1, bgr,
                        W00a, W00b, b00r, W10a, W10b, b10r,
                        Wt01, bt01r, Wt11, bt11r)
    return (y1_2d.reshape(-1), y0_2d.reshape(-1), phi_x)


# speedup vs baseline: 73.7905x; 1.1738x over previous
import jax
import jax.numpy as jnp
from jax import lax
from jax.experimental import pallas as pl
from jax.experimental.pallas import tpu as pltpu
from jax.experimental.pallas import tpu_sc as plsc

N = 50000
X_DIM = 128
H_DIM = 64
G_DIM = 32
E = 1600000

NC = 2
NS = 16
NW = NC * NS

CHUNK = 128
ECH = E // CHUNK
BASE_CH = ECH // NW
EXTRA = ECH - NW * BASE_CH
EXTRA_BASE = NW * BASE_CH

NP = 51200
ROWS_PER_TILE = NP // NS
DROW = 16
DROWS = NP // DROW
DROWS_PER_TILE = DROWS // NS
DCH = DROWS // CHUNK

RB = 2048
GRID = NP // RB

GRP = 13
GROUPS = BASE_CH // GRP
NBUF = 3


def _tc1_body(x_ref, t_ref, w1_ref, b1_ref, wg_ref, phi_ref, xw_ref):
    x = x_ref[...]
    phi = jnp.maximum(
        jax.lax.dot_general(x, w1_ref[...], (((1,), (0,)), ((), ())),
                            preferred_element_type=jnp.float32)
        + b1_ref[...], 0.0)
    phi_ref[...] = phi
    phi_t = t_ref[...] * phi
    xw_ref[...] = jax.lax.dot_general(phi_t, wg_ref[...], (((1,), (0,)), ((), ())),
                                      preferred_element_type=jnp.float32)


def _tc1(features, t2d, W1, b1r, Wg):
    return pl.pallas_call(
        _tc1_body,
        grid=(GRID,),
        in_specs=[
            pl.BlockSpec((RB, X_DIM), lambda i: (i, 0)),
            pl.BlockSpec((RB, 1), lambda i: (i, 0)),
            pl.BlockSpec((X_DIM, H_DIM), lambda i: (0, 0)),
            pl.BlockSpec((1, H_DIM), lambda i: (0, 0)),
            pl.BlockSpec((H_DIM, G_DIM), lambda i: (0, 0)),
        ],
        out_specs=[
            pl.BlockSpec((RB, H_DIM), lambda i: (i, 0)),
            pl.BlockSpec((RB, G_DIM), lambda i: (i, 0)),
        ],
        out_shape=[
            jax.ShapeDtypeStruct((N, H_DIM), jnp.float32),
            jax.ShapeDtypeStruct((N, G_DIM), jnp.float32),
        ],
    )(features, t2d, W1, b1r, Wg)


def _sca_body(ei_hbm, zeros16_hbm, idx_hbm, out_hbm,
              dst_v, dst_x, hist2, idx_v, deg_sh):
    c = lax.axis_index("c")
    s = lax.axis_index("s")
    w = c * NS + s
    start = w * BASE_CH
    pltpu.sync_copy(ei_hbm.at[1].at[pl.ds(start, BASE_CH)], dst_v)
    pltpu.sync_copy(idx_hbm, idx_v)
    pltpu.sync_copy(zeros16_hbm,
                    deg_sh.at[pl.ds(s * DROWS_PER_TILE, DROWS_PER_TILE)])

    z16 = jnp.zeros((16,), jnp.float32)

    def zero_body(i, carry):
        hist2[i, pl.ds(0, 16)] = z16
        return carry

    lax.fori_loop(0, DROWS, zero_body, 0)

    ones = jnp.ones((16,), jnp.float32)

    def hist_vec(idx):
        hi = lax.shift_right_logical(idx, 4)
        lo = lax.bitwise_and(idx, 15)
        plsc.addupdate_scatter(hist2, [hi, lo], ones)

    def chunk_body(j, carry):
        for v in range(CHUNK // 16):
            hist_vec(dst_v[j, pl.ds(v * 16, 16)])
        return carry

    lax.fori_loop(0, BASE_CH, chunk_body, 0)

    @pl.when(w < EXTRA)
    def _():
        pltpu.sync_copy(ei_hbm.at[1].at[pl.ds(EXTRA_BASE + w, 1)], dst_x)
        for v in range(CHUNK // 16):
            hist_vec(dst_x[0, pl.ds(v * 16, 16)])

    plsc.subcore_barrier()

    def red_body(k, carry):
        pltpu.sync_copy(hist2.at[pl.ds(k * CHUNK, CHUNK)],
                        deg_sh.at[idx_v.at[k]], add=True)
        return carry

    lax.fori_loop(0, DCH, red_body, 0)
    plsc.subcore_barrier()

    dsl = pl.ds(s * DROWS_PER_TILE, DROWS_PER_TILE)
    pltpu.sync_copy(deg_sh.at[dsl], out_hbm.at[c].at[dsl])


def _sca(ei3, zeros16, idx_rows):
    mesh = plsc.VectorSubcoreMesh(core_axis_name="c", subcore_axis_name="s")
    return pl.kernel(
        _sca_body,
        out_type=jax.ShapeDtypeStruct((NC, DROWS, DROW), jnp.float32),
        mesh=mesh,
        compiler_params=pltpu.CompilerParams(needs_layout_passes=False,
                                             use_tc_tiling_on_sc=False),
        scratch_types=[
            pltpu.VMEM((BASE_CH, CHUNK), jnp.int32),
            pltpu.VMEM((1, CHUNK), jnp.int32),
            pltpu.VMEM((DROWS, DROW), jnp.float32),
            pltpu.VMEM((DCH, CHUNK), jnp.int32),
            pltpu.VMEM_SHARED((DROWS, DROW), jnp.float32),
        ],
    )(ei3, zeros16, idx_rows)


def _tc2_body(degp_ref, xw_ref, ones_ref, xs_ref, dinv_ref):
    degsum = jax.lax.dot_general(degp_ref[...], ones_ref[...],
                                 (((0,), (0,)), ((), ())),
                                 preferred_element_type=jnp.float32)
    dinv = jax.lax.rsqrt(degsum + 1.0)
    dinv_ref[...] = dinv
    xs_ref[...] = xw_ref[...] * dinv


def _tc2(deg2, xw, ones_col):
    return pl.pallas_call(
        _tc2_body,
        grid=(GRID,),
        in_specs=[
            pl.BlockSpec((NC, RB), lambda i: (0, i)),
            pl.BlockSpec((RB, G_DIM), lambda i: (i, 0)),
            pl.BlockSpec((NC, 1), lambda i: (0, 0)),
        ],
        out_specs=[
            pl.BlockSpec((RB, G_DIM), lambda i: (i, 0)),
            pl.BlockSpec((RB, 1), lambda i: (i, 0)),
        ],
        out_shape=[
            jax.ShapeDtypeStruct((N, G_DIM), jnp.float32),
            jax.ShapeDtypeStruct((N, 1), jnp.float32),
        ],
    )(deg2, xw, ones_col)


def _scb_body(xs_hbm, ei_hbm, zeros_hbm, out0, out1,
              src_v, dst_v, src_x, dst_x, rows0, rows1, rows2,
              gsem0, gsem1, gsem2, ssem0, ssem1, ssem2, acc):
    c = lax.axis_index("c")
    s = lax.axis_index("s")
    w = c * NS + s
    base = s * ROWS_PER_TILE
    sl = pl.ds(base, ROWS_PER_TILE)
    pltpu.sync_copy(zeros_hbm, acc.at[sl])
    plsc.subcore_barrier()

    rows = (rows0, rows1, rows2)
    gsem = (gsem0, gsem1, gsem2)
    ssem = (ssem0, ssem1, ssem2)
    start = w * BASE_CH

    def group_body(g, carry):
        gsl = pl.ds(start + g * GRP, GRP)
        pltpu.sync_copy(ei_hbm.at[0].at[gsl], src_v)
        pltpu.sync_copy(ei_hbm.at[1].at[gsl], dst_v)
        gath = [None] * GRP
        for b in range(NBUF):
            gath[b] = pltpu.async_copy(xs_hbm.at[src_v.at[b]], rows[b], gsem[b])
        tail = []
        for j in range(GRP):
            b = j % NBUF
            gath[j].wait()
            scat = pltpu.async_copy(rows[b], acc.at[dst_v.at[j]],
                                    ssem[b], add=True)
            if j + NBUF < GRP:
                scat.wait()
                gath[j + NBUF] = pltpu.async_copy(
                    xs_hbm.at[src_v.at[j + NBUF]], rows[b], gsem[b])
            else:
                tail.append(scat)
        for scat in tail:
            scat.wait()
        return carry

    lax.fori_loop(0, GROUPS, group_body, 0)

    @pl.when(w < EXTRA)
    def _():
        xsl = pl.ds(EXTRA_BASE + w, 1)
        pltpu.sync_copy(ei_hbm.at[0].at[xsl], src_x)
        pltpu.sync_copy(ei_hbm.at[1].at[xsl], dst_x)
        pltpu.async_copy(xs_hbm.at[src_x.at[0]], rows0, gsem0).wait()
        pltpu.sync_copy(rows0, acc.at[dst_x.at[0]], add=True)

    plsc.subcore_barrier()

    @pl.when(c == 0)
    def _():
        pltpu.sync_copy(acc.at[sl], out0.at[sl])

    @pl.when(c == 1)
    def _():
        pltpu.sync_copy(acc.at[sl], out1.at[sl])


def _scb(xs, ei3, zeros_blk):
    mesh = plsc.VectorSubcoreMesh(core_axis_name="c", subcore_axis_name="s")
    return pl.kernel(
        _scb_body,
        out_type=[
            jax.ShapeDtypeStruct((NP, G_DIM), jnp.float32),
            jax.ShapeDtypeStruct((NP, G_DIM), jnp.float32),
        ],
        mesh=mesh,
        compiler_params=pltpu.CompilerParams(use_tc_tiling_on_sc=False),
        scratch_types=[
            pltpu.VMEM((GRP, CHUNK), jnp.int32),
            pltpu.VMEM((GRP, CHUNK), jnp.int32),
            pltpu.VMEM((1, CHUNK), jnp.int32),
            pltpu.VMEM((1, CHUNK), jnp.int32),
            pltpu.VMEM((CHUNK, G_DIM), jnp.float32),
            pltpu.VMEM((CHUNK, G_DIM), jnp.float32),
            pltpu.VMEM((CHUNK, G_DIM), jnp.float32),
            pltpu.SemaphoreType.DMA,
            pltpu.SemaphoreType.DMA,
            pltpu.SemaphoreType.DMA,
            pltpu.SemaphoreType.DMA,
            pltpu.SemaphoreType.DMA,
            pltpu.SemaphoreType.DMA,
            pltpu.VMEM_SHARED((NP, G_DIM), jnp.float32),
        ],
    )(xs, ei3, zeros_blk)


def _tc3_body(phi_ref, xs_ref, dinv_ref, a0_ref, a1_ref, bg_ref,
              w00a_ref, w00b_ref, b00_ref, w10a_ref, w10b_ref, b10_ref,
              wt0_ref, bt0_ref, wt1_ref, bt1_ref, y0_ref, y1_ref):
    gnn = dinv_ref[...] * (a0_ref[...] + a1_ref[...] + xs_ref[...]) + bg_ref[...]
    phi = phi_ref[...]

    def head(wa, wb, b, wt, bt):
        y = jnp.maximum(
            jax.lax.dot_general(phi, wa, (((1,), (0,)), ((), ())),
                                preferred_element_type=jnp.float32)
            + jax.lax.dot_general(gnn, wb, (((1,), (0,)), ((), ())),
                                  preferred_element_type=jnp.float32)
            + b, 0.0)
        return jax.lax.dot_general(y, wt, (((1,), (0,)), ((), ())),
                                   preferred_element_type=jnp.float32) + bt

    y0_ref[...] = head(w00a_ref[...], w00b_ref[...], b00_ref[...],
                       wt0_ref[...], bt0_ref[...])
    y1_ref[...] = head(w10a_ref[...], w10b_ref[...], b10_ref[...],
                       wt1_ref[...], bt1_ref[...])


def _tc3(phi_x, xs, dinv, acc0, acc1, bgr, W00a, W00b, b00r,
         W10a, W10b, b10r, Wt01, bt01r, Wt11, bt11r):
    YR = H_DIM + G_DIM

    def full(shape):
        return pl.BlockSpec(shape, lambda i: tuple(0 for _ in shape))

    return pl.pallas_call(
        _tc3_body,
        grid=(GRID,),
        in_specs=[
            pl.BlockSpec((RB, H_DIM), lambda i: (i, 0)),
            pl.BlockSpec((RB, G_DIM), lambda i: (i, 0)),
            pl.BlockSpec((RB, 1), lambda i: (i, 0)),
            pl.BlockSpec((RB, G_DIM), lambda i: (i, 0)),
            pl.BlockSpec((RB, G_DIM), lambda i: (i, 0)),
            full((1, G_DIM)),
            full((H_DIM, YR)), full((G_DIM, YR)), full((1, YR)),
            full((H_DIM, YR)), full((G_DIM, YR)), full((1, YR)),
            full((YR, 1)), full((1, 1)),
            full((YR, 1)), full((1, 1)),
        ],
        out_specs=[
            pl.BlockSpec((RB, 1), lambda i: (i, 0)),
            pl.BlockSpec((RB, 1), lambda i: (i, 0)),
        ],
        out_shape=[
            jax.ShapeDtypeStruct((N, 1), jnp.float32),
            jax.ShapeDtypeStruct((N, 1), jnp.float32),
        ],
    )(phi_x, xs, dinv, acc0, acc1, bgr, W00a, W00b, b00r,
      W10a, W10b, b10r, Wt01, bt01r, Wt11, bt11r)


def kernel(features, treatments, edge_index, W1, b1, Wg, bg,
           W00, b00, W10, b10, Wt01, bt01, Wt11, bt11):
    t2d = treatments.reshape(N, 1)
    b1r = b1.reshape(1, H_DIM)
    bgr = bg.reshape(1, G_DIM)
    b00r = b00.reshape(1, -1)
    b10r = b10.reshape(1, -1)
    bt01r = bt01.reshape(1, 1)
    bt11r = bt11.reshape(1, 1)
    W00a, W00b = W00[:H_DIM], W00[H_DIM:]
    W10a, W10b = W10[:H_DIM], W10[H_DIM:]

    ei3 = edge_index.reshape(2, ECH, CHUNK)
    zeros_blk = jnp.zeros((ROWS_PER_TILE, G_DIM), jnp.float32)
    zeros16 = jnp.zeros((DROWS_PER_TILE, DROW), jnp.float32)
    idx_rows = jnp.arange(DROWS, dtype=jnp.int32).reshape(DCH, CHUNK)
    ones_col = jnp.ones((NC, 1), jnp.float32)

    phi_x, xw = _tc1(features, t2d, W1, b1r, Wg)
    deg2 = _sca(ei3, zeros16, idx_rows).reshape(NC, NP)
    xs, dinv = _tc2(deg2, xw, ones_col)
    acc0, acc1 = _scb(xs, ei3, zeros_blk)
    y0_2d, y1_2d = _tc3(phi_x, xs, dinv, acc0, acc1, bgr,
                        W00a, W00b, b00r, W10a, W10b, b10r,
                        Wt01, bt01r, Wt11, bt11r)
    return (y1_2d.reshape(-1), y0_2d.reshape(-1), phi_x)


# speedup vs baseline: 79.7454x; 1.0807x over previous
import jax
import jax.numpy as jnp
from jax import lax
from jax.experimental import pallas as pl
from jax.experimental.pallas import tpu as pltpu
from jax.experimental.pallas import tpu_sc as plsc

N = 50000
X_DIM = 128
H_DIM = 64
G_DIM = 32
E = 1600000

NC = 2
NS = 16
NW = NC * NS

CHUNK = 128
ECH = E // CHUNK
BASE_CH = ECH // NW
EXTRA = ECH - NW * BASE_CH
EXTRA_BASE = NW * BASE_CH

NP = 51200
ROWS_PER_TILE = NP // NS
DROW = 16
DROWS = NP // DROW
DROWS_PER_TILE = DROWS // NS
DCH = DROWS // CHUNK

RB = 2048
GRID = NP // RB

GRP = 15
GROUPS = BASE_CH // GRP
NBUF = 5


def _tc1_body(x_ref, t_ref, w1_ref, b1_ref, wg_ref, phi_ref, xw_ref):
    x = x_ref[...]
    phi = jnp.maximum(
        jax.lax.dot_general(x, w1_ref[...], (((1,), (0,)), ((), ())),
                            preferred_element_type=jnp.float32)
        + b1_ref[...], 0.0)
    phi_ref[...] = phi
    phi_t = t_ref[...] * phi
    xw_ref[...] = jax.lax.dot_general(phi_t, wg_ref[...], (((1,), (0,)), ((), ())),
                                      preferred_element_type=jnp.float32)


def _tc1(features, t2d, W1, b1r, Wg):
    return pl.pallas_call(
        _tc1_body,
        grid=(GRID,),
        in_specs=[
            pl.BlockSpec((RB, X_DIM), lambda i: (i, 0)),
            pl.BlockSpec((RB, 1), lambda i: (i, 0)),
            pl.BlockSpec((X_DIM, H_DIM), lambda i: (0, 0)),
            pl.BlockSpec((1, H_DIM), lambda i: (0, 0)),
            pl.BlockSpec((H_DIM, G_DIM), lambda i: (0, 0)),
        ],
        out_specs=[
            pl.BlockSpec((RB, H_DIM), lambda i: (i, 0)),
            pl.BlockSpec((RB, G_DIM), lambda i: (i, 0)),
        ],
        out_shape=[
            jax.ShapeDtypeStruct((N, H_DIM), jnp.float32),
            jax.ShapeDtypeStruct((N, G_DIM), jnp.float32),
        ],
    )(features, t2d, W1, b1r, Wg)


def _sca_body(ei_hbm, zeros16_hbm, idx_hbm, out_hbm,
              dst_v, dst_x, hist2, idx_v, deg_sh):
    c = lax.axis_index("c")
    s = lax.axis_index("s")
    w = c * NS + s
    start = w * BASE_CH
    pltpu.sync_copy(ei_hbm.at[1].at[pl.ds(start, BASE_CH)], dst_v)
    pltpu.sync_copy(idx_hbm, idx_v)
    pltpu.sync_copy(zeros16_hbm,
                    deg_sh.at[pl.ds(s * DROWS_PER_TILE, DROWS_PER_TILE)])

    z16 = jnp.zeros((16,), jnp.float32)

    def zero_body(i, carry):
        hist2[i, pl.ds(0, 16)] = z16
        return carry

    lax.fori_loop(0, DROWS, zero_body, 0)

    ones = jnp.ones((16,), jnp.float32)

    def hist_vec(idx):
        hi = lax.shift_right_logical(idx, 4)
        lo = lax.bitwise_and(idx, 15)
        plsc.addupdate_scatter(hist2, [hi, lo], ones)

    def chunk_body(j, carry):
        for v in range(CHUNK // 16):
            hist_vec(dst_v[j, pl.ds(v * 16, 16)])
        return carry

    lax.fori_loop(0, BASE_CH, chunk_body, 0)

    @pl.when(w < EXTRA)
    def _():
        pltpu.sync_copy(ei_hbm.at[1].at[pl.ds(EXTRA_BASE + w, 1)], dst_x)
        for v in range(CHUNK // 16):
            hist_vec(dst_x[0, pl.ds(v * 16, 16)])

    plsc.subcore_barrier()

    def red_body(k, carry):
        pltpu.sync_copy(hist2.at[pl.ds(k * CHUNK, CHUNK)],
                        deg_sh.at[idx_v.at[k]], add=True)
        return carry

    lax.fori_loop(0, DCH, red_body, 0)
    plsc.subcore_barrier()

    dsl = pl.ds(s * DROWS_PER_TILE, DROWS_PER_TILE)
    pltpu.sync_copy(deg_sh.at[dsl], out_hbm.at[c].at[dsl])


def _sca(ei3, zeros16, idx_rows):
    mesh = plsc.VectorSubcoreMesh(core_axis_name="c", subcore_axis_name="s")
    return pl.kernel(
        _sca_body,
        out_type=jax.ShapeDtypeStruct((NC, DROWS, DROW), jnp.float32),
        mesh=mesh,
        compiler_params=pltpu.CompilerParams(needs_layout_passes=False,
                                             use_tc_tiling_on_sc=False),
        scratch_types=[
            pltpu.VMEM((BASE_CH, CHUNK), jnp.int32),
            pltpu.VMEM((1, CHUNK), jnp.int32),
            pltpu.VMEM((DROWS, DROW), jnp.float32),
            pltpu.VMEM((DCH, CHUNK), jnp.int32),
            pltpu.VMEM_SHARED((DROWS, DROW), jnp.float32),
        ],
    )(ei3, zeros16, idx_rows)


def _tc2_body(degp_ref, xw_ref, ones_ref, xs_ref, dinv_ref):
    degsum = jax.lax.dot_general(degp_ref[...], ones_ref[...],
                                 (((0,), (0,)), ((), ())),
                                 preferred_element_type=jnp.float32)
    dinv = jax.lax.rsqrt(degsum + 1.0)
    dinv_ref[...] = dinv
    xs_ref[...] = xw_ref[...] * dinv


def _tc2(deg2, xw, ones_col):
    return pl.pallas_call(
        _tc2_body,
        grid=(GRID,),
        in_specs=[
            pl.BlockSpec((NC, RB), lambda i: (0, i)),
            pl.BlockSpec((RB, G_DIM), lambda i: (i, 0)),
            pl.BlockSpec((NC, 1), lambda i: (0, 0)),
        ],
        out_specs=[
            pl.BlockSpec((RB, G_DIM), lambda i: (i, 0)),
            pl.BlockSpec((RB, 1), lambda i: (i, 0)),
        ],
        out_shape=[
            jax.ShapeDtypeStruct((N, G_DIM), jnp.float32),
            jax.ShapeDtypeStruct((N, 1), jnp.float32),
        ],
    )(deg2, xw, ones_col)


def _scb_body(xs_hbm, ei_hbm, zeros_hbm, out0, out1,
              src_v, dst_v, src_x, dst_x, rows0, rows1, rows2, rows3, rows4,
              gsem0, gsem1, gsem2, gsem3, gsem4,
              ssem0, ssem1, ssem2, ssem3, ssem4, acc):
    c = lax.axis_index("c")
    s = lax.axis_index("s")
    w = c * NS + s
    base = s * ROWS_PER_TILE
    sl = pl.ds(base, ROWS_PER_TILE)
    pltpu.sync_copy(zeros_hbm, acc.at[sl])
    plsc.subcore_barrier()

    rows = (rows0, rows1, rows2, rows3, rows4)
    gsem = (gsem0, gsem1, gsem2, gsem3, gsem4)
    ssem = (ssem0, ssem1, ssem2, ssem3, ssem4)
    start = w * BASE_CH

    def group_body(g, carry):
        gsl = pl.ds(start + g * GRP, GRP)
        pltpu.sync_copy(ei_hbm.at[0].at[gsl], src_v)
        pltpu.sync_copy(ei_hbm.at[1].at[gsl], dst_v)
        gath = [None] * GRP
        for b in range(NBUF):
            gath[b] = pltpu.async_copy(xs_hbm.at[src_v.at[b]], rows[b], gsem[b])
        tail = []
        for j in range(GRP):
            b = j % NBUF
            gath[j].wait()
            scat = pltpu.async_copy(rows[b], acc.at[dst_v.at[j]],
                                    ssem[b], add=True)
            if j + NBUF < GRP:
                scat.wait()
                gath[j + NBUF] = pltpu.async_copy(
                    xs_hbm.at[src_v.at[j + NBUF]], rows[b], gsem[b])
            else:
                tail.append(scat)
        for scat in tail:
            scat.wait()
        return carry

    lax.fori_loop(0, GROUPS, group_body, 0)

    @pl.when(w < EXTRA)
    def _():
        xsl = pl.ds(EXTRA_BASE + w, 1)
        pltpu.sync_copy(ei_hbm.at[0].at[xsl], src_x)
        pltpu.sync_copy(ei_hbm.at[1].at[xsl], dst_x)
        pltpu.async_copy(xs_hbm.at[src_x.at[0]], rows0, gsem0).wait()
        pltpu.sync_copy(rows0, acc.at[dst_x.at[0]], add=True)

    plsc.subcore_barrier()

    @pl.when(c == 0)
    def _():
        pltpu.sync_copy(acc.at[sl], out0.at[sl])

    @pl.when(c == 1)
    def _():
        pltpu.sync_copy(acc.at[sl], out1.at[sl])


def _scb(xs, ei3, zeros_blk):
    mesh = plsc.VectorSubcoreMesh(core_axis_name="c", subcore_axis_name="s")
    return pl.kernel(
        _scb_body,
        out_type=[
            jax.ShapeDtypeStruct((NP, G_DIM), jnp.float32),
            jax.ShapeDtypeStruct((NP, G_DIM), jnp.float32),
        ],
        mesh=mesh,
        compiler_params=pltpu.CompilerParams(use_tc_tiling_on_sc=False),
        scratch_types=[
            pltpu.VMEM((GRP, CHUNK), jnp.int32),
            pltpu.VMEM((GRP, CHUNK), jnp.int32),
            pltpu.VMEM((1, CHUNK), jnp.int32),
            pltpu.VMEM((1, CHUNK), jnp.int32),
            pltpu.VMEM((CHUNK, G_DIM), jnp.float32),
            pltpu.VMEM((CHUNK, G_DIM), jnp.float32),
            pltpu.VMEM((CHUNK, G_DIM), jnp.float32),
            pltpu.VMEM((CHUNK, G_DIM), jnp.float32),
            pltpu.VMEM((CHUNK, G_DIM), jnp.float32),
            pltpu.SemaphoreType.DMA,
            pltpu.SemaphoreType.DMA,
            pltpu.SemaphoreType.DMA,
            pltpu.SemaphoreType.DMA,
            pltpu.SemaphoreType.DMA,
            pltpu.SemaphoreType.DMA,
            pltpu.SemaphoreType.DMA,
            pltpu.SemaphoreType.DMA,
            pltpu.SemaphoreType.DMA,
            pltpu.SemaphoreType.DMA,
            pltpu.VMEM_SHARED((NP, G_DIM), jnp.float32),
        ],
    )(xs, ei3, zeros_blk)


def _tc3_body(phi_ref, xs_ref, dinv_ref, a0_ref, a1_ref, bg_ref,
              w00a_ref, w00b_ref, b00_ref, w10a_ref, w10b_ref, b10_ref,
              wt0_ref, bt0_ref, wt1_ref, bt1_ref, y0_ref, y1_ref):
    gnn = dinv_ref[...] * (a0_ref[...] + a1_ref[...] + xs_ref[...]) + bg_ref[...]
    phi = phi_ref[...]

    def head(wa, wb, b, wt, bt):
        y = jnp.maximum(
            jax.lax.dot_general(phi, wa, (((1,), (0,)), ((), ())),
                                preferred_element_type=jnp.float32)
            + jax.lax.dot_general(gnn, wb, (((1,), (0,)), ((), ())),
                                  preferred_element_type=jnp.float32)
            + b, 0.0)
        return jax.lax.dot_general(y, wt, (((1,), (0,)), ((), ())),
                                   preferred_element_type=jnp.float32) + bt

    y0_ref[...] = head(w00a_ref[...], w00b_ref[...], b00_ref[...],
                       wt0_ref[...], bt0_ref[...])
    y1_ref[...] = head(w10a_ref[...], w10b_ref[...], b10_ref[...],
                       wt1_ref[...], bt1_ref[...])


def _tc3(phi_x, xs, dinv, acc0, acc1, bgr, W00a, W00b, b00r,
         W10a, W10b, b10r, Wt01, bt01r, Wt11, bt11r):
    YR = H_DIM + G_DIM

    def full(shape):
        return pl.BlockSpec(shape, lambda i: tuple(0 for _ in shape))

    return pl.pallas_call(
        _tc3_body,
        grid=(GRID,),
        in_specs=[
            pl.BlockSpec((RB, H_DIM), lambda i: (i, 0)),
            pl.BlockSpec((RB, G_DIM), lambda i: (i, 0)),
            pl.BlockSpec((RB, 1), lambda i: (i, 0)),
            pl.BlockSpec((RB, G_DIM), lambda i: (i, 0)),
            pl.BlockSpec((RB, G_DIM), lambda i: (i, 0)),
            full((1, G_DIM)),
            full((H_DIM, YR)), full((G_DIM, YR)), full((1, YR)),
            full((H_DIM, YR)), full((G_DIM, YR)), full((1, YR)),
            full((YR, 1)), full((1, 1)),
            full((YR, 1)), full((1, 1)),
        ],
        out_specs=[
            pl.BlockSpec((RB, 1), lambda i: (i, 0)),
            pl.BlockSpec((RB, 1), lambda i: (i, 0)),
        ],
        out_shape=[
            jax.ShapeDtypeStruct((N, 1), jnp.float32),
            jax.ShapeDtypeStruct((N, 1), jnp.float32),
        ],
    )(phi_x, xs, dinv, acc0, acc1, bgr, W00a, W00b, b00r,
      W10a, W10b, b10r, Wt01, bt01r, Wt11, bt11r)


def kernel(features, treatments, edge_index, W1, b1, Wg, bg,
           W00, b00, W10, b10, Wt01, bt01, Wt11, bt11):
    t2d = treatments.reshape(N, 1)
    b1r = b1.reshape(1, H_DIM)
    bgr = bg.reshape(1, G_DIM)
    b00r = b00.reshape(1, -1)
    b10r = b10.reshape(1, -1)
    bt01r = bt01.reshape(1, 1)
    bt11r = bt11.reshape(1, 1)
    W00a, W00b = W00[:H_DIM], W00[H_DIM:]
    W10a, W10b = W10[:H_DIM], W10[H_DIM:]

    ei3 = edge_index.reshape(2, ECH, CHUNK)
    zeros_blk = jnp.zeros((ROWS_PER_TILE, G_DIM), jnp.float32)
    zeros16 = jnp.zeros((DROWS_PER_TILE, DROW), jnp.float32)
    idx_rows = jnp.arange(DROWS, dtype=jnp.int32).reshape(DCH, CHUNK)
    ones_col = jnp.ones((NC, 1), jnp.float32)

    phi_x, xw = _tc1(features, t2d, W1, b1r, Wg)
    deg2 = _sca(ei3, zeros16, idx_rows).reshape(NC, NP)
    xs, dinv = _tc2(deg2, xw, ones_col)
    acc0, acc1 = _scb(xs, ei3, zeros_blk)
    y0_2d, y1_2d = _tc3(phi_x, xs, dinv, acc0, acc1, bgr,
                        W00a, W00b, b00r, W10a, W10b, b10r,
                        Wt01, bt01r, Wt11, bt11r)
    return (y1_2d.reshape(-1), y0_2d.reshape(-1), phi_x)
